# Initial kernel scaffold; baseline (speedup 1.0000x reference)
import jax
import jax.numpy as jnp
from jax.experimental import pallas as pl


def kernel(x, edge_index, edge_weight, W, b):
    raise NotImplementedError("write your pallas kernel here")


# speedup vs baseline: 2.7743x; 2.7743x over previous
import functools

import jax
import jax.numpy as jnp
from jax import lax
from jax.experimental import pallas as pl
from jax.experimental.compute_on import compute_on
from jax.experimental.pallas import tpu as pltpu
from jax.experimental.pallas import tpu_sc as plsc

N = 10000
E = 160000
D = 256
H = 128
K = 5

NC = 2
NS = 16
L = 16
EB = 80
EPT = E // NS
NBATCH = EPT // EB
WCH = 80
NCHUNK = N // WCH
CPT = -(-NCHUNK // NS)
KH = K * NC
RB = 400

_MESH = plsc.VectorSubcoreMesh(core_axis_name="c", subcore_axis_name="s")
_SC_PARAMS = pltpu.CompilerParams(needs_layout_passes=False)


def _zero_vmem_rows(buf, rows, width):
    def body(i, _):
        for t in range(width // L):
            buf[i, pl.ds(t * L, L)] = jnp.zeros((L,), jnp.float32)
        return 0
    lax.fori_loop(0, rows, body, 0)


def _deg_body(row_hbm, w_hbm, deg_hbm, row_v, w_v, wsp_v, wb_v, acc_sh):
    c = lax.axis_index("c")
    s = lax.axis_index("s")
    _zero_vmem_rows(wb_v, WCH, H)
    for j in range(CPT):
        idx = j * NS + s

        @pl.when(idx < NCHUNK)
        def _():
            pltpu.sync_copy(wb_v, acc_sh.at[pl.ds(idx * WCH, WCH)])
    plsc.subcore_barrier()

    def batch(i, _):
        base = s * EPT + i * EB
        pltpu.sync_copy(row_hbm.at[pl.ds(base, EB)], row_v)
        pltpu.sync_copy(w_hbm.at[pl.ds(base, EB)], w_v)

        def edge(e, _):
            wv = plsc.load_gather(w_v, [jnp.full((L,), e, jnp.int32)])
            for t in range(H // L):
                wsp_v[e, pl.ds(t * L, L)] = wv
            return 0
        lax.fori_loop(0, EB, edge, 0)
        pltpu.sync_copy(wsp_v, acc_sh.at[row_v], add=True)
        return 0
    lax.fori_loop(0, NBATCH, batch, 0)
    plsc.subcore_barrier()

    @pl.when(c == 0)
    def _():
        for j in range(CPT):
            idx = j * NS + s

            @pl.when(idx < NCHUNK)
            def _():
                pltpu.sync_copy(acc_sh.at[pl.ds(idx * WCH, WCH)], wb_v)
                pltpu.sync_copy(wb_v, deg_hbm.at[pl.ds(idx * WCH, WCH)])


def _make_deg(interpret=False):
    return functools.partial(
        pl.kernel,
        out_type=jax.ShapeDtypeStruct((N, H), jnp.float32),
        mesh=_MESH,
        scratch_types=[
            pltpu.VMEM((EB,), jnp.int32),
            pltpu.VMEM((EB,), jnp.float32),
            pltpu.VMEM((EB, H), jnp.float32),
            pltpu.VMEM((WCH, H), jnp.float32),
            pltpu.VMEM_SHARED((N, H), jnp.float32),
        ],
        compiler_params=_SC_PARAMS,
        interpret=interpret,
    )(_deg_body)


_deg_kernel = _make_deg()


def _dinv_body(deg_ref, dinv_ref):
    deg = jnp.max(deg_ref[...], axis=1)
    dinv_ref[...] = jnp.where(deg > 0, lax.rsqrt(deg), 0.0)


def _dinv(deg16):
    return pl.pallas_call(
        _dinv_body,
        out_shape=jax.ShapeDtypeStruct((N,), jnp.float32),
    )(deg16)


NORM_B = 400


@functools.partial(
    pl.kernel,
    out_type=jax.ShapeDtypeStruct((E,), jnp.float32),
    mesh=_MESH,
    scratch_types=[
        pltpu.VMEM((N,), jnp.float32),
        pltpu.VMEM((NORM_B,), jnp.int32),
        pltpu.VMEM((NORM_B,), jnp.int32),
        pltpu.VMEM((NORM_B,), jnp.float32),
        pltpu.VMEM((NORM_B,), jnp.float32),
    ],
    compiler_params=_SC_PARAMS,
)
def _norm_kernel(row_hbm, col_hbm, w_hbm, dinv_hbm, norm_hbm,
                 dinv_v, row_v, col_v, w_v, out_v):
    c = lax.axis_index("c")
    s = lax.axis_index("s")
    pltpu.sync_copy(dinv_hbm, dinv_v)

    def batch(i, _):
        base = s * EPT + i * NORM_B
        pltpu.sync_copy(row_hbm.at[pl.ds(base, NORM_B)], row_v)
        pltpu.sync_copy(col_hbm.at[pl.ds(base, NORM_B)], col_v)
        pltpu.sync_copy(w_hbm.at[pl.ds(base, NORM_B)], w_v)
        for g in range(NORM_B // L):
            sl = pl.ds(g * L, L)
            dr = plsc.load_gather(dinv_v, [row_v[sl]])
            dc = plsc.load_gather(dinv_v, [col_v[sl]])
            out_v[sl] = -(w_v[sl] * dr * dc)

        @pl.when(c == 0)
        def _():
            pltpu.sync_copy(out_v, norm_hbm.at[pl.ds(base, NORM_B)])
        return 0
    lax.fori_loop(0, EPT // NORM_B, batch, 0)


def _prop_body(scale, xs_hbm, row_hbm, col_hbm, norm_hbm, sub_hbm, out_hbm,
               row_v, col_v, gcol_v, norm_v, msg_v, wb_v, sb_v, acc_sh, sem):
    c = lax.axis_index("c")
    s = lax.axis_index("s")
    coff = c * N

    _zero_vmem_rows(wb_v, WCH, H)
    for j in range(CPT):
        idx = j * NS + s

        @pl.when(idx < NCHUNK)
        def _():
            pltpu.sync_copy(wb_v, acc_sh.at[pl.ds(idx * WCH, WCH)])
    plsc.subcore_barrier()

    def batch(i, _):
        base = s * EPT + i * EB
        pltpu.sync_copy(row_hbm.at[pl.ds(base, EB)], row_v)
        pltpu.sync_copy(col_hbm.at[pl.ds(base, EB)], col_v)
        pltpu.sync_copy(norm_hbm.at[pl.ds(base, EB)], norm_v)
        for g in range(EB // L):
            sl = pl.ds(g * L, L)
            gcol_v[sl] = col_v[sl] + coff
            norm_v[sl] = norm_v[sl] * scale
        pltpu.async_copy(xs_hbm.at[gcol_v], msg_v, sem).wait()

        def edge(e, _):
            nb = plsc.load_gather(norm_v, [jnp.full((L,), e, jnp.int32)])
            for t in range(H // L):
                sl = pl.ds(t * L, L)
                msg_v[e, sl] = msg_v[e, sl] * nb
            return 0
        lax.fori_loop(0, EB, edge, 0)
        pltpu.sync_copy(msg_v, acc_sh.at[row_v], add=True)
        return 0
    lax.fori_loop(0, NBATCH, batch, 0)
    plsc.subcore_barrier()

    for j in range(CPT):
        idx = j * NS + s

        @pl.when(idx < NCHUNK)
        def _():
            r0 = idx * WCH
            pltpu.sync_copy(acc_sh.at[pl.ds(r0, WCH)], wb_v)
            pltpu.sync_copy(sub_hbm.at[pl.ds(coff + r0, WCH)], sb_v)

            def wsub(i, _):
                for t in range(H // L):
                    sl = pl.ds(t * L, L)
                    wb_v[i, sl] = wb_v[i, sl] - sb_v[i, sl]
                return 0
            lax.fori_loop(0, WCH, wsub, 0)
            pltpu.sync_copy(wb_v, out_hbm.at[pl.ds(coff + r0, WCH)])


def _make_prop(scale, interpret=False):
    return functools.partial(
        pl.kernel,
        out_type=jax.ShapeDtypeStruct((NC * N, H), jnp.float32),
        mesh=_MESH,
        scratch_types=[
            pltpu.VMEM((EB,), jnp.int32),
            pltpu.VMEM((EB,), jnp.int32),
            pltpu.VMEM((EB,), jnp.int32),
            pltpu.VMEM((EB,), jnp.float32),
            pltpu.VMEM((EB, H), jnp.float32),
            pltpu.VMEM((WCH, H), jnp.float32),
            pltpu.VMEM((WCH, H), jnp.float32),
            pltpu.VMEM_SHARED((N, H), jnp.float32),
            pltpu.SemaphoreType.DMA,
        ],
        compiler_params=_SC_PARAMS,
        interpret=interpret,
    )(functools.partial(_prop_body, scale))


_prop1 = _make_prop(1.0)
_prop2 = _make_prop(2.0)


def _mm_body(xs_ref, w_ref, b_ref, out_ref):
    kh = pl.program_id(1)

    @pl.when(kh == 0)
    def _():
        out_ref[...] = jnp.zeros_like(out_ref)

    out_ref[...] += jnp.dot(xs_ref[0], w_ref[0],
                            preferred_element_type=jnp.float32)

    @pl.when(kh == KH - 1)
    def _():
        out_ref[...] += b_ref[...]


def _matmul(xs_stack, w_stack, b2d):
    return pl.pallas_call(
        _mm_body,
        grid=(N // RB, KH),
        in_specs=[
            pl.BlockSpec((1, RB, H), lambda r, k: (k, r, 0)),
            pl.BlockSpec((1, H, D), lambda r, k: (k, 0, 0)),
            pl.BlockSpec((1, D), lambda r, k: (0, 0)),
        ],
        out_specs=pl.BlockSpec((RB, D), lambda r, k: (r, 0)),
        out_shape=jax.ShapeDtypeStruct((N, D), jnp.float32),
        compiler_params=pltpu.CompilerParams(
            dimension_semantics=("parallel", "arbitrary")),
    )(xs_stack, w_stack, b2d)


def kernel(x, edge_index, edge_weight, W, b):
    row = edge_index[0]
    col = edge_index[1]
    xs = x.reshape(N, NC, H).transpose(1, 0, 2).reshape(NC * N, H)

    with compute_on("tpu_sparsecore"):
        deg16 = _deg_kernel(row, edge_weight)
    dinv = _dinv(deg16)
    with compute_on("tpu_sparsecore"):
        norm = _norm_kernel(row, col, edge_weight, dinv)

    tx = [xs]
    zeros = jnp.zeros_like(xs)
    with compute_on("tpu_sparsecore"):
        tx.append(_prop1(xs, row, col, norm, zeros))
    for _ in range(2, K):
        with compute_on("tpu_sparsecore"):
            tx.append(_prop2(tx[-1], row, col, norm, tx[-2]))

    tx = list(lax.optimization_barrier(tuple(tx)))
    xs_stack = jnp.stack(tx).reshape(KH, N, H)
    w_stack = W.reshape(K, NC, H, D).reshape(KH, H, D)
    return _matmul(xs_stack, w_stack, b.reshape(1, D))


# speedup vs baseline: 5.1066x; 1.8407x over previous
import functools

import jax
import jax.numpy as jnp
from jax import lax
from jax.experimental import pallas as pl
from jax.experimental.compute_on import compute_on
from jax.experimental.pallas import tpu as pltpu
from jax.experimental.pallas import tpu_sc as plsc

N = 10000
E = 160000
D = 256
H = 128
K = 5

NC = 2
NS = 16
L = 16
EB = 80
EPT = E // NS
NBATCH = EPT // EB
WCH = 80
NCHUNK = N // WCH
CPT = -(-NCHUNK // NS)
KH = K * NC
RB = 400

_MESH = plsc.VectorSubcoreMesh(core_axis_name="c", subcore_axis_name="s")
_SC_PARAMS = pltpu.CompilerParams(needs_layout_passes=False)


def _zero_vmem_rows(buf, rows, width):
    def body(i, _):
        for t in range(width // L):
            buf[i, pl.ds(t * L, L)] = jnp.zeros((L,), jnp.float32)
        return 0
    lax.fori_loop(0, rows, body, 0)


def _deg_body(row_hbm, w_hbm, deg_hbm, row_v, w_v, wsp_v, wb_v, acc_sh):
    c = lax.axis_index("c")
    s = lax.axis_index("s")
    _zero_vmem_rows(wb_v, WCH, H)
    for j in range(CPT):
        idx = j * NS + s

        @pl.when(idx < NCHUNK)
        def _():
            pltpu.sync_copy(wb_v, acc_sh.at[pl.ds(idx * WCH, WCH)])
    plsc.subcore_barrier()

    def batch(i, _):
        base = s * EPT + i * EB
        pltpu.sync_copy(row_hbm.at[pl.ds(base, EB)], row_v)
        pltpu.sync_copy(w_hbm.at[pl.ds(base, EB)], w_v)

        def edge(e, _):
            wv = plsc.load_gather(w_v, [jnp.full((L,), e, jnp.int32)])
            for t in range(H // L):
                wsp_v[e, pl.ds(t * L, L)] = wv
            return 0
        lax.fori_loop(0, EB, edge, 0)
        pltpu.sync_copy(wsp_v, acc_sh.at[row_v], add=True)
        return 0
    lax.fori_loop(0, NBATCH, batch, 0)
    plsc.subcore_barrier()

    @pl.when(c == 0)
    def _():
        for j in range(CPT):
            idx = j * NS + s

            @pl.when(idx < NCHUNK)
            def _():
                pltpu.sync_copy(acc_sh.at[pl.ds(idx * WCH, WCH)], wb_v)
                pltpu.sync_copy(wb_v, deg_hbm.at[pl.ds(idx * WCH, WCH)])


def _make_deg(interpret=False):
    return functools.partial(
        pl.kernel,
        out_type=jax.ShapeDtypeStruct((N, H), jnp.float32),
        mesh=_MESH,
        scratch_types=[
            pltpu.VMEM((EB,), jnp.int32),
            pltpu.VMEM((EB,), jnp.float32),
            pltpu.VMEM((EB, H), jnp.float32),
            pltpu.VMEM((WCH, H), jnp.float32),
            pltpu.VMEM_SHARED((N, H), jnp.float32),
        ],
        compiler_params=_SC_PARAMS,
        interpret=interpret,
    )(_deg_body)


_deg_kernel = _make_deg()


def _dinv_body(deg_ref, dinv_ref):
    deg = jnp.max(deg_ref[...], axis=1)
    dinv_ref[...] = jnp.where(deg > 0, lax.rsqrt(deg), 0.0)


def _dinv(deg16):
    return pl.pallas_call(
        _dinv_body,
        out_shape=jax.ShapeDtypeStruct((N,), jnp.float32),
    )(deg16)


NORM_B = 400


@functools.partial(
    pl.kernel,
    out_type=jax.ShapeDtypeStruct((E,), jnp.float32),
    mesh=_MESH,
    scratch_types=[
        pltpu.VMEM((N,), jnp.float32),
        pltpu.VMEM((NORM_B,), jnp.int32),
        pltpu.VMEM((NORM_B,), jnp.int32),
        pltpu.VMEM((NORM_B,), jnp.float32),
        pltpu.VMEM((NORM_B,), jnp.float32),
    ],
    compiler_params=_SC_PARAMS,
)
def _norm_kernel(row_hbm, col_hbm, w_hbm, dinv_hbm, norm_hbm,
                 dinv_v, row_v, col_v, w_v, out_v):
    c = lax.axis_index("c")
    s = lax.axis_index("s")
    pltpu.sync_copy(dinv_hbm, dinv_v)

    def batch(i, _):
        base = s * EPT + i * NORM_B
        pltpu.sync_copy(row_hbm.at[pl.ds(base, NORM_B)], row_v)
        pltpu.sync_copy(col_hbm.at[pl.ds(base, NORM_B)], col_v)
        pltpu.sync_copy(w_hbm.at[pl.ds(base, NORM_B)], w_v)
        for g in range(NORM_B // L):
            sl = pl.ds(g * L, L)
            dr = plsc.load_gather(dinv_v, [row_v[sl]])
            dc = plsc.load_gather(dinv_v, [col_v[sl]])
            out_v[sl] = -(w_v[sl] * dr * dc)

        @pl.when(c == 0)
        def _():
            pltpu.sync_copy(out_v, norm_hbm.at[pl.ds(base, NORM_B)])
        return 0
    lax.fori_loop(0, EPT // NORM_B, batch, 0)


HB0 = (NBATCH + 1) // 2
EPT2 = HB0 * EB


def _prop_body(scale, xs_hbm, row_hbm, col_hbm, norm_hbm, sub_hbm, out_hbm,
               rowL_v, colL_v, normL_v, row0_v, row1_v, gcol0_v, gcol1_v,
               nsc0_v, nsc1_v, msg0_v, msg1_v, wb_v, acc_sh, sem0, sem1):
    c = lax.axis_index("c")
    s = lax.axis_index("s")
    coff = c * N
    bufs = ((row0_v, gcol0_v, nsc0_v, msg0_v, sem0),
            (row1_v, gcol1_v, nsc1_v, msg1_v, sem1))

    for j in range(CPT):
        idx = j * NS + s

        @pl.when(idx < NCHUNK)
        def _():
            r0 = idx * WCH
            pltpu.sync_copy(sub_hbm.at[pl.ds(coff + r0, WCH)], wb_v)

            def neg(i, _):
                for t in range(H // L):
                    sl = pl.ds(t * L, L)
                    wb_v[i, sl] = -wb_v[i, sl]
                return 0
            lax.fori_loop(0, WCH, neg, 0)
            pltpu.sync_copy(wb_v, acc_sh.at[pl.ds(r0, WCH)])
    plsc.subcore_barrier()

    def start(i, k):
        rowb, gcol, nsc, msg, sem = bufs[k]
        for g in range(EB // L):
            sl = pl.ds(g * L, L)
            src = pl.ds(i * EB + g * L, L)
            rowb[sl] = rowL_v[src]
            gcol[sl] = colL_v[src] + coff
            nsc[sl] = normL_v[src] * scale
        pltpu.async_copy(xs_hbm.at[gcol], msg, sem)

    def process(i, k):
        rowb, gcol, nsc, msg, sem = bufs[k]
        pltpu.make_async_copy(xs_hbm.at[gcol], msg, sem).wait()

        def edge(e, _):
            nb = plsc.load_gather(nsc, [jnp.full((L,), e, jnp.int32)])
            for t in range(H // L):
                sl = pl.ds(t * L, L)
                msg[e, sl] = msg[e, sl] * nb
            return 0
        lax.fori_loop(0, EB, edge, 0)
        pltpu.sync_copy(msg, acc_sh.at[rowb], add=True)

    for half in range(2):
        hb = HB0 if half == 0 else NBATCH - HB0
        base = s * EPT + half * EPT2
        ne = hb * EB
        pltpu.sync_copy(row_hbm.at[pl.ds(base, ne)], rowL_v.at[pl.ds(0, ne)])
        pltpu.sync_copy(col_hbm.at[pl.ds(base, ne)], colL_v.at[pl.ds(0, ne)])
        pltpu.sync_copy(norm_hbm.at[pl.ds(base, ne)], normL_v.at[pl.ds(0, ne)])
        start(0, 0)

        def batch(i, _):
            @pl.when(i % 2 == 0)
            def _():
                @pl.when(i + 1 < hb)
                def _():
                    start(i + 1, 1)
                process(i, 0)

            @pl.when(i % 2 == 1)
            def _():
                @pl.when(i + 1 < hb)
                def _():
                    start(i + 1, 0)
                process(i, 1)
            return 0
        lax.fori_loop(0, hb, batch, 0)
    plsc.subcore_barrier()

    for j in range(CPT):
        idx = j * NS + s

        @pl.when(idx < NCHUNK)
        def _():
            r0 = idx * WCH
            pltpu.sync_copy(acc_sh.at[pl.ds(r0, WCH)], wb_v)
            pltpu.sync_copy(wb_v, out_hbm.at[pl.ds(coff + r0, WCH)])


def _make_prop(scale, interpret=False):
    return functools.partial(
        pl.kernel,
        out_type=jax.ShapeDtypeStruct((NC * N, H), jnp.float32),
        mesh=_MESH,
        scratch_types=[
            pltpu.VMEM((EPT2,), jnp.int32),
            pltpu.VMEM((EPT2,), jnp.int32),
            pltpu.VMEM((EPT2,), jnp.float32),
            pltpu.VMEM((EB,), jnp.int32),
            pltpu.VMEM((EB,), jnp.int32),
            pltpu.VMEM((EB,), jnp.int32),
            pltpu.VMEM((EB,), jnp.int32),
            pltpu.VMEM((EB,), jnp.float32),
            pltpu.VMEM((EB,), jnp.float32),
            pltpu.VMEM((EB, H), jnp.float32),
            pltpu.VMEM((EB, H), jnp.float32),
            pltpu.VMEM((WCH, H), jnp.float32),
            pltpu.VMEM_SHARED((N, H), jnp.float32),
            pltpu.SemaphoreType.DMA,
            pltpu.SemaphoreType.DMA,
        ],
        compiler_params=_SC_PARAMS,
        interpret=interpret,
    )(functools.partial(_prop_body, scale))


_prop1 = _make_prop(1.0)
_prop2 = _make_prop(2.0)


def _mm_body(xs_ref, w_ref, b_ref, out_ref):
    kh = pl.program_id(1)

    @pl.when(kh == 0)
    def _():
        out_ref[...] = jnp.zeros_like(out_ref)

    out_ref[...] += jnp.dot(xs_ref[0], w_ref[0],
                            preferred_element_type=jnp.float32)

    @pl.when(kh == KH - 1)
    def _():
        out_ref[...] += b_ref[...]


def _matmul(xs_stack, w_stack, b2d):
    return pl.pallas_call(
        _mm_body,
        grid=(N // RB, KH),
        in_specs=[
            pl.BlockSpec((1, RB, H), lambda r, k: (k, r, 0)),
            pl.BlockSpec((1, H, D), lambda r, k: (k, 0, 0)),
            pl.BlockSpec((1, D), lambda r, k: (0, 0)),
        ],
        out_specs=pl.BlockSpec((RB, D), lambda r, k: (r, 0)),
        out_shape=jax.ShapeDtypeStruct((N, D), jnp.float32),
        compiler_params=pltpu.CompilerParams(
            dimension_semantics=("parallel", "arbitrary")),
    )(xs_stack, w_stack, b2d)


def kernel(x, edge_index, edge_weight, W, b):
    row = edge_index[0]
    col = edge_index[1]
    xs = x.reshape(N, NC, H).transpose(1, 0, 2).reshape(NC * N, H)

    with compute_on("tpu_sparsecore"):
        deg16 = _deg_kernel(row, edge_weight)
    dinv = _dinv(deg16)
    with compute_on("tpu_sparsecore"):
        norm = _norm_kernel(row, col, edge_weight, dinv)

    tx = [xs]
    zeros = jnp.zeros_like(xs)
    with compute_on("tpu_sparsecore"):
        tx.append(_prop1(xs, row, col, norm, zeros))
    for _ in range(2, K):
        with compute_on("tpu_sparsecore"):
            tx.append(_prop2(tx[-1], row, col, norm, tx[-2]))

    tx = list(lax.optimization_barrier(tuple(tx)))
    xs_stack = jnp.stack(tx).reshape(KH, N, H)
    w_stack = W.reshape(K, NC, H, D).reshape(KH, H, D)
    return _matmul(xs_stack, w_stack, b.reshape(1, D))


# speedup vs baseline: 5.6844x; 1.1131x over previous
import functools

import jax
import jax.numpy as jnp
from jax import lax
from jax.experimental import pallas as pl
from jax.experimental.compute_on import compute_on
from jax.experimental.pallas import tpu as pltpu
from jax.experimental.pallas import tpu_sc as plsc

N = 10000
E = 160000
D = 256
H = 128
K = 5

NC = 2
NS = 16
L = 16
EB = 80
EPT = E // NS
NBATCH = EPT // EB
WCH = 80
NCHUNK = N // WCH
CPT = -(-NCHUNK // NS)
KH = K * NC
RB = 400

_MESH = plsc.VectorSubcoreMesh(core_axis_name="c", subcore_axis_name="s")
_SC_PARAMS = pltpu.CompilerParams(needs_layout_passes=False)


def _zero_vmem_rows(buf, rows, width):
    def body(i, _):
        for t in range(width // L):
            buf[i, pl.ds(t * L, L)] = jnp.zeros((L,), jnp.float32)
        return 0
    lax.fori_loop(0, rows, body, 0)


def _deg_body(row_hbm, w_hbm, deg_hbm, row_v, w_v, wsp_v, wb_v, acc_sh):
    c = lax.axis_index("c")
    s = lax.axis_index("s")
    _zero_vmem_rows(wb_v, WCH, H)
    for j in range(CPT):
        idx = j * NS + s

        @pl.when(idx < NCHUNK)
        def _():
            pltpu.sync_copy(wb_v, acc_sh.at[pl.ds(idx * WCH, WCH)])
    plsc.subcore_barrier()

    def batch(j, _):
        i = 2 * j + c

        @pl.when(i < NBATCH)
        def _():
            base = s * EPT + i * EB
            pltpu.sync_copy(row_hbm.at[pl.ds(base, EB)], row_v)
            pltpu.sync_copy(w_hbm.at[pl.ds(base, EB)], w_v)

            def edge(e, _):
                wv = plsc.load_gather(w_v, [jnp.full((L,), e, jnp.int32)])
                for t in range(H // L):
                    wsp_v[e, pl.ds(t * L, L)] = wv
                return 0
            lax.fori_loop(0, EB, edge, 0)
            pltpu.sync_copy(wsp_v, acc_sh.at[row_v], add=True)
        return 0
    lax.fori_loop(0, (NBATCH + 1) // 2, batch, 0)
    plsc.subcore_barrier()

    for j in range(CPT):
        idx = j * NS + s

        @pl.when(idx < NCHUNK)
        def _():
            pltpu.sync_copy(acc_sh.at[pl.ds(idx * WCH, WCH)], wb_v)
            pltpu.sync_copy(wb_v, deg_hbm.at[pl.ds(c * N + idx * WCH, WCH)])


def _make_deg(interpret=False):
    return functools.partial(
        pl.kernel,
        out_type=jax.ShapeDtypeStruct((NC * N, H), jnp.float32),
        mesh=_MESH,
        scratch_types=[
            pltpu.VMEM((EB,), jnp.int32),
            pltpu.VMEM((EB,), jnp.float32),
            pltpu.VMEM((EB, H), jnp.float32),
            pltpu.VMEM((WCH, H), jnp.float32),
            pltpu.VMEM_SHARED((N, H), jnp.float32),
        ],
        compiler_params=_SC_PARAMS,
        interpret=interpret,
    )(_deg_body)


_deg_kernel = _make_deg()


def _dinv_body(deg_ref, dinv_ref):
    d = deg_ref[...]
    deg = jnp.max(d[:N], axis=1) + jnp.max(d[N:], axis=1)
    dinv_ref[...] = jnp.where(deg > 0, lax.rsqrt(deg), 0.0)


def _dinv(deg16):
    return pl.pallas_call(
        _dinv_body,
        out_shape=jax.ShapeDtypeStruct((N,), jnp.float32),
    )(deg16)


NORM_B = 400


@functools.partial(
    pl.kernel,
    out_type=jax.ShapeDtypeStruct((E,), jnp.float32),
    mesh=_MESH,
    scratch_types=[
        pltpu.VMEM((N,), jnp.float32),
        pltpu.VMEM((NORM_B,), jnp.int32),
        pltpu.VMEM((NORM_B,), jnp.int32),
        pltpu.VMEM((NORM_B,), jnp.float32),
        pltpu.VMEM((NORM_B,), jnp.float32),
    ],
    compiler_params=_SC_PARAMS,
)
def _norm_kernel(row_hbm, col_hbm, w_hbm, dinv_hbm, norm_hbm,
                 dinv_v, row_v, col_v, w_v, out_v):
    c = lax.axis_index("c")
    s = lax.axis_index("s")
    pltpu.sync_copy(dinv_hbm, dinv_v)

    def batch(j, _):
        i = 2 * j + c

        @pl.when(i < EPT // NORM_B)
        def _():
            base = s * EPT + i * NORM_B
            pltpu.sync_copy(row_hbm.at[pl.ds(base, NORM_B)], row_v)
            pltpu.sync_copy(col_hbm.at[pl.ds(base, NORM_B)], col_v)
            pltpu.sync_copy(w_hbm.at[pl.ds(base, NORM_B)], w_v)
            for g in range(NORM_B // L):
                sl = pl.ds(g * L, L)
                dr = plsc.load_gather(dinv_v, [row_v[sl]])
                dc = plsc.load_gather(dinv_v, [col_v[sl]])
                out_v[sl] = -(w_v[sl] * dr * dc)
            pltpu.sync_copy(out_v, norm_hbm.at[pl.ds(base, NORM_B)])
        return 0
    lax.fori_loop(0, (EPT // NORM_B + 1) // 2, batch, 0)


HB0 = (NBATCH + 1) // 2
EPT2 = HB0 * EB


def _prop_body(scale, xs_hbm, row_hbm, col_hbm, norm_hbm, sub_hbm, out_hbm,
               rowL_v, colL_v, normL_v, row0_v, row1_v, gcol0_v, gcol1_v,
               nsc0_v, nsc1_v, msg0_v, msg1_v, wb_v, acc_sh, sem0, sem1):
    c = lax.axis_index("c")
    s = lax.axis_index("s")
    coff = c * N
    bufs = ((row0_v, gcol0_v, nsc0_v, msg0_v, sem0),
            (row1_v, gcol1_v, nsc1_v, msg1_v, sem1))

    for j in range(CPT):
        idx = j * NS + s

        @pl.when(idx < NCHUNK)
        def _():
            r0 = idx * WCH
            pltpu.sync_copy(sub_hbm.at[pl.ds(coff + r0, WCH)], wb_v)

            def neg(i, _):
                for t in range(H // L):
                    sl = pl.ds(t * L, L)
                    wb_v[i, sl] = -wb_v[i, sl]
                return 0
            lax.fori_loop(0, WCH, neg, 0)
            pltpu.sync_copy(wb_v, acc_sh.at[pl.ds(r0, WCH)])
    plsc.subcore_barrier()

    def start(i, k):
        rowb, gcol, nsc, msg, sem = bufs[k]
        for g in range(EB // L):
            sl = pl.ds(g * L, L)
            src = pl.ds(i * EB + g * L, L)
            rowb[sl] = rowL_v[src]
            gcol[sl] = colL_v[src] + coff
            nsc[sl] = normL_v[src] * scale
        pltpu.async_copy(xs_hbm.at[gcol], msg, sem)

    def process(i, k):
        rowb, gcol, nsc, msg, sem = bufs[k]
        pltpu.make_async_copy(xs_hbm.at[gcol], msg, sem).wait()

        def edge(e, _):
            nb = plsc.load_gather(nsc, [jnp.full((L,), e, jnp.int32)])
            for t in range(H // L):
                sl = pl.ds(t * L, L)
                msg[e, sl] = msg[e, sl] * nb
            return 0
        lax.fori_loop(0, EB, edge, 0)
        pltpu.sync_copy(msg, acc_sh.at[rowb], add=True)

    for half in range(2):
        hb = HB0 if half == 0 else NBATCH - HB0
        base = s * EPT + half * EPT2
        ne = hb * EB
        pltpu.sync_copy(row_hbm.at[pl.ds(base, ne)], rowL_v.at[pl.ds(0, ne)])
        pltpu.sync_copy(col_hbm.at[pl.ds(base, ne)], colL_v.at[pl.ds(0, ne)])
        pltpu.sync_copy(norm_hbm.at[pl.ds(base, ne)], normL_v.at[pl.ds(0, ne)])
        start(0, 0)

        def batch(i, _):
            @pl.when(i % 2 == 0)
            def _():
                @pl.when(i + 1 < hb)
                def _():
                    start(i + 1, 1)
                process(i, 0)

            @pl.when(i % 2 == 1)
            def _():
                @pl.when(i + 1 < hb)
                def _():
                    start(i + 1, 0)
                process(i, 1)
            return 0
        lax.fori_loop(0, hb, batch, 0)
    plsc.subcore_barrier()

    for j in range(CPT):
        idx = j * NS + s

        @pl.when(idx < NCHUNK)
        def _():
            r0 = idx * WCH
            pltpu.sync_copy(acc_sh.at[pl.ds(r0, WCH)], wb_v)
            pltpu.sync_copy(wb_v, out_hbm.at[pl.ds(coff + r0, WCH)])


def _make_prop(scale, interpret=False):
    return functools.partial(
        pl.kernel,
        out_type=jax.ShapeDtypeStruct((NC * N, H), jnp.float32),
        mesh=_MESH,
        scratch_types=[
            pltpu.VMEM((EPT2,), jnp.int32),
            pltpu.VMEM((EPT2,), jnp.int32),
            pltpu.VMEM((EPT2,), jnp.float32),
            pltpu.VMEM((EB,), jnp.int32),
            pltpu.VMEM((EB,), jnp.int32),
            pltpu.VMEM((EB,), jnp.int32),
            pltpu.VMEM((EB,), jnp.int32),
            pltpu.VMEM((EB,), jnp.float32),
            pltpu.VMEM((EB,), jnp.float32),
            pltpu.VMEM((EB, H), jnp.float32),
            pltpu.VMEM((EB, H), jnp.float32),
            pltpu.VMEM((WCH, H), jnp.float32),
            pltpu.VMEM_SHARED((N, H), jnp.float32),
            pltpu.SemaphoreType.DMA,
            pltpu.SemaphoreType.DMA,
        ],
        compiler_params=_SC_PARAMS,
        interpret=interpret,
    )(functools.partial(_prop_body, scale))


_prop1 = _make_prop(1.0)
_prop2 = _make_prop(2.0)


def _mm_body(xs_ref, w_ref, b_ref, out_ref):
    kh = pl.program_id(1)

    @pl.when(kh == 0)
    def _():
        out_ref[...] = jnp.zeros_like(out_ref)

    out_ref[...] += jnp.dot(xs_ref[0], w_ref[0],
                            preferred_element_type=jnp.float32)

    @pl.when(kh == KH - 1)
    def _():
        out_ref[...] += b_ref[...]


def _matmul(xs_stack, w_stack, b2d):
    return pl.pallas_call(
        _mm_body,
        grid=(N // RB, KH),
        in_specs=[
            pl.BlockSpec((1, RB, H), lambda r, k: (k, r, 0)),
            pl.BlockSpec((1, H, D), lambda r, k: (k, 0, 0)),
            pl.BlockSpec((1, D), lambda r, k: (0, 0)),
        ],
        out_specs=pl.BlockSpec((RB, D), lambda r, k: (r, 0)),
        out_shape=jax.ShapeDtypeStruct((N, D), jnp.float32),
        compiler_params=pltpu.CompilerParams(
            dimension_semantics=("parallel", "arbitrary")),
    )(xs_stack, w_stack, b2d)


def kernel(x, edge_index, edge_weight, W, b):
    row = edge_index[0]
    col = edge_index[1]
    xs = x.reshape(N, NC, H).transpose(1, 0, 2).reshape(NC * N, H)

    with compute_on("tpu_sparsecore"):
        deg16 = _deg_kernel(row, edge_weight)
    dinv = _dinv(deg16)
    with compute_on("tpu_sparsecore"):
        norm = _norm_kernel(row, col, edge_weight, dinv)

    tx = [xs]
    zeros = jnp.zeros_like(xs)
    with compute_on("tpu_sparsecore"):
        tx.append(_prop1(xs, row, col, norm, zeros))
    for _ in range(2, K):
        with compute_on("tpu_sparsecore"):
            tx.append(_prop2(tx[-1], row, col, norm, tx[-2]))

    tx = list(lax.optimization_barrier(tuple(tx)))
    xs_stack = jnp.stack(tx).reshape(KH, N, H)
    w_stack = W.reshape(K, NC, H, D).reshape(KH, H, D)
    return _matmul(xs_stack, w_stack, b.reshape(1, D))


# speedup vs baseline: 5.8355x; 1.0266x over previous
import functools

import jax
import jax.numpy as jnp
from jax import lax
from jax.experimental import pallas as pl
from jax.experimental.compute_on import compute_on
from jax.experimental.pallas import tpu as pltpu
from jax.experimental.pallas import tpu_sc as plsc

N = 10000
E = 160000
D = 256
H = 128
K = 5

NC = 2
NS = 16
L = 16
EB = 80
EPT = E // NS
NBATCH = EPT // EB
WCH = 80
NCHUNK = N // WCH
CPT = -(-NCHUNK // NS)
KH = K * NC
RB = 400

_MESH = plsc.VectorSubcoreMesh(core_axis_name="c", subcore_axis_name="s")
_SC_PARAMS = pltpu.CompilerParams(needs_layout_passes=False)


def _zero_vmem_rows(buf, rows, width):
    def body(i, _):
        for t in range(width // L):
            buf[i, pl.ds(t * L, L)] = jnp.zeros((L,), jnp.float32)
        return 0
    lax.fori_loop(0, rows, body, 0)


def _deg_body(row_hbm, w_hbm, deg_hbm, row_v, w_v, wsp_v, wb_v, acc_sh):
    c = lax.axis_index("c")
    s = lax.axis_index("s")
    _zero_vmem_rows(wb_v, WCH, H)
    for j in range(CPT):
        idx = j * NS + s

        @pl.when(idx < NCHUNK)
        def _():
            pltpu.sync_copy(wb_v, acc_sh.at[pl.ds(idx * WCH, WCH)])
    plsc.subcore_barrier()

    def batch(j, _):
        i = 2 * j + c

        @pl.when(i < NBATCH)
        def _():
            base = s * EPT + i * EB
            pltpu.sync_copy(row_hbm.at[pl.ds(base, EB)], row_v)
            pltpu.sync_copy(w_hbm.at[pl.ds(base, EB)], w_v)

            def edge8(i8, _):
                for u in range(8):
                    e = i8 * 8 + u
                    wv = plsc.load_gather(w_v, [jnp.full((L,), e, jnp.int32)])
                    for t in range(H // L):
                        wsp_v[e, pl.ds(t * L, L)] = wv
                return 0
            lax.fori_loop(0, EB // 8, edge8, 0)
            pltpu.sync_copy(wsp_v, acc_sh.at[row_v], add=True)
        return 0
    lax.fori_loop(0, (NBATCH + 1) // 2, batch, 0)
    plsc.subcore_barrier()

    for j in range(CPT):
        idx = j * NS + s

        @pl.when(idx < NCHUNK)
        def _():
            pltpu.sync_copy(acc_sh.at[pl.ds(idx * WCH, WCH)], wb_v)
            pltpu.sync_copy(wb_v, deg_hbm.at[pl.ds(c * N + idx * WCH, WCH)])


def _make_deg(interpret=False):
    return functools.partial(
        pl.kernel,
        out_type=jax.ShapeDtypeStruct((NC * N, H), jnp.float32),
        mesh=_MESH,
        scratch_types=[
            pltpu.VMEM((EB,), jnp.int32),
            pltpu.VMEM((EB,), jnp.float32),
            pltpu.VMEM((EB, H), jnp.float32),
            pltpu.VMEM((WCH, H), jnp.float32),
            pltpu.VMEM_SHARED((N, H), jnp.float32),
        ],
        compiler_params=_SC_PARAMS,
        interpret=interpret,
    )(_deg_body)


_deg_kernel = _make_deg()


def _dinv_body(deg_ref, dinv_ref):
    d = deg_ref[...]
    deg = jnp.max(d[:N], axis=1) + jnp.max(d[N:], axis=1)
    dinv_ref[...] = jnp.where(deg > 0, lax.rsqrt(deg), 0.0)


def _dinv(deg16):
    return pl.pallas_call(
        _dinv_body,
        out_shape=jax.ShapeDtypeStruct((N,), jnp.float32),
    )(deg16)


NORM_B = 400


@functools.partial(
    pl.kernel,
    out_type=jax.ShapeDtypeStruct((E,), jnp.float32),
    mesh=_MESH,
    scratch_types=[
        pltpu.VMEM((N,), jnp.float32),
        pltpu.VMEM((NORM_B,), jnp.int32),
        pltpu.VMEM((NORM_B,), jnp.int32),
        pltpu.VMEM((NORM_B,), jnp.float32),
        pltpu.VMEM((NORM_B,), jnp.float32),
    ],
    compiler_params=_SC_PARAMS,
)
def _norm_kernel(row_hbm, col_hbm, w_hbm, dinv_hbm, norm_hbm,
                 dinv_v, row_v, col_v, w_v, out_v):
    c = lax.axis_index("c")
    s = lax.axis_index("s")
    pltpu.sync_copy(dinv_hbm, dinv_v)

    def batch(j, _):
        i = 2 * j + c

        @pl.when(i < EPT // NORM_B)
        def _():
            base = s * EPT + i * NORM_B
            pltpu.sync_copy(row_hbm.at[pl.ds(base, NORM_B)], row_v)
            pltpu.sync_copy(col_hbm.at[pl.ds(base, NORM_B)], col_v)
            pltpu.sync_copy(w_hbm.at[pl.ds(base, NORM_B)], w_v)
            for g in range(NORM_B // L):
                sl = pl.ds(g * L, L)
                dr = plsc.load_gather(dinv_v, [row_v[sl]])
                dc = plsc.load_gather(dinv_v, [col_v[sl]])
                out_v[sl] = -(w_v[sl] * dr * dc)
            pltpu.sync_copy(out_v, norm_hbm.at[pl.ds(base, NORM_B)])
        return 0
    lax.fori_loop(0, (EPT // NORM_B + 1) // 2, batch, 0)


HB0 = (NBATCH + 1) // 2
EPT2 = HB0 * EB


def _prop_body(scale, xs_hbm, row_hbm, col_hbm, norm_hbm, sub_hbm, out_hbm,
               rowL_v, colL_v, normL_v, row0_v, row1_v, gcol0_v, gcol1_v,
               nsc0_v, nsc1_v, msg0_v, msg1_v, wb_v, acc_sh, sem0, sem1):
    c = lax.axis_index("c")
    s = lax.axis_index("s")
    coff = c * N
    bufs = ((row0_v, gcol0_v, nsc0_v, msg0_v, sem0),
            (row1_v, gcol1_v, nsc1_v, msg1_v, sem1))

    for j in range(CPT):
        idx = j * NS + s

        @pl.when(idx < NCHUNK)
        def _():
            r0 = idx * WCH
            pltpu.sync_copy(sub_hbm.at[pl.ds(coff + r0, WCH)], wb_v)

            def neg(i, _):
                for t in range(H // L):
                    sl = pl.ds(t * L, L)
                    wb_v[i, sl] = -wb_v[i, sl]
                return 0
            lax.fori_loop(0, WCH, neg, 0)
            pltpu.sync_copy(wb_v, acc_sh.at[pl.ds(r0, WCH)])
    plsc.subcore_barrier()

    def start(i, k):
        rowb, gcol, nsc, msg, sem = bufs[k]
        for g in range(EB // L):
            sl = pl.ds(g * L, L)
            src = pl.ds(i * EB + g * L, L)
            rowb[sl] = rowL_v[src]
            gcol[sl] = colL_v[src] + coff
            nsc[sl] = normL_v[src] * scale
        pltpu.async_copy(xs_hbm.at[gcol], msg, sem)

    def process(i, k):
        rowb, gcol, nsc, msg, sem = bufs[k]
        pltpu.make_async_copy(xs_hbm.at[gcol], msg, sem).wait()

        def edge8(i8, _):
            for u in range(8):
                e = i8 * 8 + u
                nb = plsc.load_gather(nsc, [jnp.full((L,), e, jnp.int32)])
                for t in range(H // L):
                    sl = pl.ds(t * L, L)
                    msg[e, sl] = msg[e, sl] * nb
            return 0
        lax.fori_loop(0, EB // 8, edge8, 0)
        pltpu.sync_copy(msg, acc_sh.at[rowb], add=True)

    for half in range(2):
        hb = HB0 if half == 0 else NBATCH - HB0
        base = s * EPT + half * EPT2
        ne = hb * EB
        pltpu.sync_copy(row_hbm.at[pl.ds(base, ne)], rowL_v.at[pl.ds(0, ne)])
        pltpu.sync_copy(col_hbm.at[pl.ds(base, ne)], colL_v.at[pl.ds(0, ne)])
        pltpu.sync_copy(norm_hbm.at[pl.ds(base, ne)], normL_v.at[pl.ds(0, ne)])
        start(0, 0)

        def batch(i, _):
            @pl.when(i % 2 == 0)
            def _():
                @pl.when(i + 1 < hb)
                def _():
                    start(i + 1, 1)
                process(i, 0)

            @pl.when(i % 2 == 1)
            def _():
                @pl.when(i + 1 < hb)
                def _():
                    start(i + 1, 0)
                process(i, 1)
            return 0
        lax.fori_loop(0, hb, batch, 0)
    plsc.subcore_barrier()

    for j in range(CPT):
        idx = j * NS + s

        @pl.when(idx < NCHUNK)
        def _():
            r0 = idx * WCH
            pltpu.sync_copy(acc_sh.at[pl.ds(r0, WCH)], wb_v)
            pltpu.sync_copy(wb_v, out_hbm.at[pl.ds(coff + r0, WCH)])


def _make_prop(scale, interpret=False):
    return functools.partial(
        pl.kernel,
        out_type=jax.ShapeDtypeStruct((NC * N, H), jnp.float32),
        mesh=_MESH,
        scratch_types=[
            pltpu.VMEM((EPT2,), jnp.int32),
            pltpu.VMEM((EPT2,), jnp.int32),
            pltpu.VMEM((EPT2,), jnp.float32),
            pltpu.VMEM((EB,), jnp.int32),
            pltpu.VMEM((EB,), jnp.int32),
            pltpu.VMEM((EB,), jnp.int32),
            pltpu.VMEM((EB,), jnp.int32),
            pltpu.VMEM((EB,), jnp.float32),
            pltpu.VMEM((EB,), jnp.float32),
            pltpu.VMEM((EB, H), jnp.float32),
            pltpu.VMEM((EB, H), jnp.float32),
            pltpu.VMEM((WCH, H), jnp.float32),
            pltpu.VMEM_SHARED((N, H), jnp.float32),
            pltpu.SemaphoreType.DMA,
            pltpu.SemaphoreType.DMA,
        ],
        compiler_params=_SC_PARAMS,
        interpret=interpret,
    )(functools.partial(_prop_body, scale))


_prop1 = _make_prop(1.0)
_prop2 = _make_prop(2.0)


def _mm_body(xs_ref, w_ref, b_ref, out_ref):
    kh = pl.program_id(1)

    @pl.when(kh == 0)
    def _():
        out_ref[...] = jnp.zeros_like(out_ref)

    out_ref[...] += jnp.dot(xs_ref[0], w_ref[0],
                            preferred_element_type=jnp.float32)

    @pl.when(kh == KH - 1)
    def _():
        out_ref[...] += b_ref[...]


def _matmul(xs_stack, w_stack, b2d):
    return pl.pallas_call(
        _mm_body,
        grid=(N // RB, KH),
        in_specs=[
            pl.BlockSpec((1, RB, H), lambda r, k: (k, r, 0)),
            pl.BlockSpec((1, H, D), lambda r, k: (k, 0, 0)),
            pl.BlockSpec((1, D), lambda r, k: (0, 0)),
        ],
        out_specs=pl.BlockSpec((RB, D), lambda r, k: (r, 0)),
        out_shape=jax.ShapeDtypeStruct((N, D), jnp.float32),
        compiler_params=pltpu.CompilerParams(
            dimension_semantics=("parallel", "arbitrary")),
    )(xs_stack, w_stack, b2d)


def kernel(x, edge_index, edge_weight, W, b):
    row = edge_index[0]
    col = edge_index[1]
    xs = x.reshape(N, NC, H).transpose(1, 0, 2).reshape(NC * N, H)

    with compute_on("tpu_sparsecore"):
        deg16 = _deg_kernel(row, edge_weight)
    dinv = _dinv(deg16)
    with compute_on("tpu_sparsecore"):
        norm = _norm_kernel(row, col, edge_weight, dinv)

    tx = [xs]
    zeros = jnp.zeros_like(xs)
    with compute_on("tpu_sparsecore"):
        tx.append(_prop1(xs, row, col, norm, zeros))
    for _ in range(2, K):
        with compute_on("tpu_sparsecore"):
            tx.append(_prop2(tx[-1], row, col, norm, tx[-2]))

    tx = list(lax.optimization_barrier(tuple(tx)))
    xs_stack = jnp.stack(tx).reshape(KH, N, H)
    w_stack = W.reshape(K, NC, H, D).reshape(KH, H, D)
    return _matmul(xs_stack, w_stack, b.reshape(1, D))
